# Initial kernel scaffold; baseline (speedup 1.0000x reference)
import jax
import jax.numpy as jnp
from jax.experimental import pallas as pl


def kernel(x, edge_index, edge_attr, batch, Wn, bn, We, be, Wm, bm, Wu, bu, Wo, bo):
    raise NotImplementedError("write your pallas kernel here")


# speedup vs baseline: 1.8634x; 1.8634x over previous
import functools

import jax
import jax.numpy as jnp
from jax import lax
from jax.experimental import pallas as pl
from jax.experimental.pallas import tpu as pltpu
from jax.experimental.pallas import tpu_sc as plsc

G = 64
HALF = 32
NS = 16
NC = 2
EBLK = 80
UROWS = 80


def _hm_body(x_ref, w_ref, b_ref, o_ref):
    y = jnp.dot(x_ref[...], w_ref[...], preferred_element_type=jnp.float32)
    y = y + b_ref[...]
    blk = y.shape[0]
    o_ref[...] = jnp.concatenate(
        [y, jnp.zeros((blk, 128 - y.shape[1]), jnp.float32)], axis=1)


def _hm_stage(xmat, w, b, blk):
    n, d = xmat.shape
    h = w.shape[1]
    return pl.pallas_call(
        _hm_body,
        grid=(n // blk,),
        in_specs=[
            pl.BlockSpec((blk, d), lambda i: (i, 0)),
            pl.BlockSpec((d, h), lambda i: (0, 0)),
            pl.BlockSpec((1, h), lambda i: (0, 0)),
        ],
        out_specs=pl.BlockSpec((blk, 128), lambda i: (i, 0)),
        out_shape=jax.ShapeDtypeStruct((n, 128), jnp.float32),
    )(xmat, w, b)


def _em_body(a_ref, b_ref, w_ref, o_ref):
    ya = jnp.dot(a_ref[...], w_ref[...], preferred_element_type=jnp.float32)
    yb = jnp.dot(b_ref[...], w_ref[...], preferred_element_type=jnp.float32)
    o_ref[...] = jnp.concatenate([ya, yb], axis=1)


def _em_stage(ea, w, blk):
    e, d = ea.shape
    h = w.shape[1]
    off = (e // 2) // blk
    return pl.pallas_call(
        _em_body,
        grid=(off,),
        in_specs=[
            pl.BlockSpec((blk, d), lambda i: (i, 0)),
            pl.BlockSpec((blk, d), lambda i: (i + off, 0)),
            pl.BlockSpec((d, h), lambda i: (0, 0)),
        ],
        out_specs=pl.BlockSpec((blk, 128), lambda i: (i, 0)),
        out_shape=jax.ShapeDtypeStruct((e // 2, 128), jnp.float32),
    )(ea, ea, w)


def _sc_edge_body(n_nodes, n_edges, hm_hbm, em_hbm, src_hbm, dst_hbm,
                  out_hbm, src_v, dst_v, xj_v, em_v, msg_v, zer_v,
                  aggr_sh, gsem, ssem):
    c = lax.axis_index("c")
    s = lax.axis_index("s")
    edges_per_tile = n_edges // NS
    nblocks = edges_per_tile // EBLK
    nunits = n_nodes // UROWS
    units_per_tile = pl.cdiv(nunits, NS)
    c32 = c * HALF

    zero16 = jnp.zeros((16,), jnp.float32)

    def zb(i, _):
        zer_v[i, 0:16] = zero16
        zer_v[i, 16:32] = zero16
        return 0

    lax.fori_loop(0, UROWS, zb, 0)

    def zs(i, _):
        u = s + i * NS

        @pl.when(u < nunits)
        def _():
            pltpu.sync_copy(zer_v, aggr_sh.at[pl.ds(u * UROWS, UROWS)])

        return 0

    lax.fori_loop(0, units_per_tile, zs, 0)
    plsc.subcore_barrier()

    ebase = s * edges_per_tile
    emoff = (s // (NS // 2)) * (n_edges // 2)
    emcol = (s // (NS // 2)) * 64 + c32

    def block(b, _):
        base = ebase + b * EBLK
        pltpu.sync_copy(src_hbm.at[pl.ds(base, EBLK)], src_v)
        pltpu.sync_copy(dst_hbm.at[pl.ds(base, EBLK)], dst_v)

        cp = pltpu.async_copy(hm_hbm.at[src_v], xj_v, gsem)
        pltpu.sync_copy(
            em_hbm.at[pl.ds(base - emoff, EBLK), pl.ds(emcol, HALF)], em_v)
        cp.wait()

        def rowfn(i, _):
            a = xj_v[i, pl.ds(c32, 16)] + em_v[i, 0:16]
            msg_v[i, 0:16] = jnp.maximum(a, 0.0)
            a = xj_v[i, pl.ds(c32 + 16, 16)] + em_v[i, 16:32]
            msg_v[i, 16:32] = jnp.maximum(a, 0.0)
            return 0

        lax.fori_loop(0, EBLK, rowfn, 0)

        pltpu.async_copy(msg_v, aggr_sh.at[dst_v], ssem, add=True).wait()
        return 0

    lax.fori_loop(0, nblocks, block, 0)
    plsc.subcore_barrier()

    def co(i, _):
        u = s + i * NS

        @pl.when(u < nunits)
        def _():
            pltpu.sync_copy(aggr_sh.at[pl.ds(u * UROWS, UROWS)],
                            out_hbm.at[pl.ds(c * n_nodes + u * UROWS,
                                             UROWS)])

        return 0

    lax.fori_loop(0, units_per_tile, co, 0)


def _sc_edge_stage(hm_pad, em_pk, src, dst, n_nodes, n_edges):
    mesh = plsc.VectorSubcoreMesh(core_axis_name="c", subcore_axis_name="s",
                                  num_cores=NC, num_subcores=NS)
    body = functools.partial(_sc_edge_body, n_nodes, n_edges)
    return pl.kernel(
        body,
        out_type=jax.ShapeDtypeStruct((NC * n_nodes, HALF), jnp.float32),
        mesh=mesh,
        compiler_params=pltpu.CompilerParams(use_tc_tiling_on_sc=False),
        scratch_types=[
            pltpu.VMEM((EBLK,), jnp.int32),
            pltpu.VMEM((EBLK,), jnp.int32),
            pltpu.VMEM((EBLK, 128), jnp.float32),
            pltpu.VMEM((EBLK, HALF), jnp.float32),
            pltpu.VMEM((EBLK, HALF), jnp.float32),
            pltpu.VMEM((UROWS, HALF), jnp.float32),
            pltpu.VMEM_SHARED((n_nodes, HALF), jnp.float32),
            pltpu.SemaphoreType.DMA,
            pltpu.SemaphoreType.DMA,
        ],
    )(hm_pad, em_pk, src, dst)


def _head_body(nblocks, a_ref, b_ref, oh_ref, wu_ref, bu_ref, wo_ref,
               bo_ref, o_ref, sums_s, cnt_s):
    i = pl.program_id(0)
    a = a_ref[...]
    b = b_ref[...]
    blk = a.shape[0]
    u = jnp.dot(a, wu_ref[:HALF, :], preferred_element_type=jnp.float32)
    u = u + jnp.dot(b, wu_ref[HALF:, :], preferred_element_type=jnp.float32)
    u = jnp.maximum(u + bu_ref[...], 0.0)

    oh = oh_ref[...]
    ps = lax.dot_general(oh, u, (((0,), (0,)), ((), ())),
                         preferred_element_type=jnp.float32)
    pc = lax.dot_general(oh, jnp.ones((blk, 1), jnp.float32),
                         (((0,), (0,)), ((), ())),
                         preferred_element_type=jnp.float32)

    @pl.when(i == 0)
    def _():
        sums_s[...] = ps
        cnt_s[...] = pc

    @pl.when(i > 0)
    def _():
        sums_s[...] = sums_s[...] + ps
        cnt_s[...] = cnt_s[...] + pc

    @pl.when(i == nblocks - 1)
    def _():
        pooled = sums_s[...] / jnp.maximum(cnt_s[...], 1.0)
        o_ref[...] = jnp.dot(pooled, wo_ref[...],
                             preferred_element_type=jnp.float32) + bo_ref[...]


def _head(aggr_a, aggr_b, oh, wu, bu2d, wo, bo2d, blk):
    n = aggr_a.shape[0]
    nblocks = n // blk
    return pl.pallas_call(
        functools.partial(_head_body, nblocks),
        grid=(nblocks,),
        in_specs=[
            pl.BlockSpec((blk, HALF), lambda i: (i, 0)),
            pl.BlockSpec((blk, HALF), lambda i: (i, 0)),
            pl.BlockSpec((blk, G), lambda i: (i, 0)),
            pl.BlockSpec((2 * HALF, G), lambda i: (0, 0)),
            pl.BlockSpec((1, G), lambda i: (0, 0)),
            pl.BlockSpec((G, 1), lambda i: (0, 0)),
            pl.BlockSpec((1, 1), lambda i: (0, 0)),
        ],
        out_specs=pl.BlockSpec((G, 1), lambda i: (0, 0)),
        out_shape=jax.ShapeDtypeStruct((G, 1), jnp.float32),
        scratch_shapes=[
            pltpu.VMEM((G, G), jnp.float32),
            pltpu.VMEM((G, 1), jnp.float32),
        ],
    )(aggr_a, aggr_b, oh, wu, bu2d, wo, bo2d)


def kernel(x, edge_index, edge_attr, batch, Wn, bn, We, be, Wm, bm, Wu, bu,
           Wo, bo):
    n, _ = x.shape
    e = edge_attr.shape[0]
    h = Wn.shape[1]

    w1 = Wn @ Wm[:h]
    w2 = We @ Wm[h:]
    beff = (bn @ Wm[:h] + be @ Wm[h:] + bm).reshape(1, h)

    hm_pad = _hm_stage(x, w1, beff, blk=10000)
    em_pk = _em_stage(edge_attr, w2, blk=10000)

    aggr = _sc_edge_stage(hm_pad, em_pk, edge_index[0], edge_index[1],
                          n, e)

    aggr_a = aggr[:n]
    aggr_b = aggr[n:]
    oh = (batch[:, None] == jnp.arange(G, dtype=batch.dtype)[None, :]
          ).astype(jnp.float32)
    out = _head(aggr_a, aggr_b, oh, Wu, bu.reshape(1, h),
                Wo, bo.reshape(1, 1), blk=10000)
    return out.reshape(G)


# speedup vs baseline: 3.0228x; 1.6222x over previous
import functools

import jax
import jax.numpy as jnp
from jax import lax
from jax.experimental import pallas as pl
from jax.experimental.pallas import tpu as pltpu
from jax.experimental.pallas import tpu_sc as plsc

G = 64
HALF = 32
NS = 16
NC = 2
EBLK = 400
ECH = 80
NCHUNK = EBLK // ECH
UROWS = 80


def _hm_body(x_ref, w_ref, b_ref, o_ref):
    y = jnp.dot(x_ref[...], w_ref[...], preferred_element_type=jnp.float32)
    y = y + b_ref[...]
    o_ref[0] = y[:, :HALF]
    o_ref[1] = y[:, HALF:]


def _hm_stage(xmat, w, b, blk):
    n, d = xmat.shape
    h = w.shape[1]
    return pl.pallas_call(
        _hm_body,
        grid=(n // blk,),
        in_specs=[
            pl.BlockSpec((blk, d), lambda i: (i, 0)),
            pl.BlockSpec((d, h), lambda i: (0, 0)),
            pl.BlockSpec((1, h), lambda i: (0, 0)),
        ],
        out_specs=pl.BlockSpec((2, blk, HALF), lambda i: (0, i, 0)),
        out_shape=jax.ShapeDtypeStruct((2, n, HALF), jnp.float32),
    )(xmat, w, b)


def _em_body(a_ref, b_ref, w_ref, o_ref):
    ya = jnp.dot(a_ref[...], w_ref[...], preferred_element_type=jnp.float32)
    yb = jnp.dot(b_ref[...], w_ref[...], preferred_element_type=jnp.float32)
    o_ref[...] = jnp.concatenate([ya, yb], axis=1)


def _em_stage(ea, w, blk):
    e, d = ea.shape
    h = w.shape[1]
    off = (e // 2) // blk
    return pl.pallas_call(
        _em_body,
        grid=(off,),
        in_specs=[
            pl.BlockSpec((blk, d), lambda i: (i, 0)),
            pl.BlockSpec((blk, d), lambda i: (i + off, 0)),
            pl.BlockSpec((d, h), lambda i: (0, 0)),
        ],
        out_specs=pl.BlockSpec((blk, 128), lambda i: (i, 0)),
        out_shape=jax.ShapeDtypeStruct((e // 2, 128), jnp.float32),
    )(ea, ea, w)


def _sc_edge_body(n_nodes, n_edges, hm_hbm, em_hbm, src_hbm, dst_hbm,
                  out_hbm, src_v, dst_v, xj_v, em_v, zer_v,
                  aggr_sh, gsem, ssem):
    c = lax.axis_index("c")
    s = lax.axis_index("s")
    edges_per_tile = n_edges // NS
    nblocks = edges_per_tile // EBLK
    nunits = n_nodes // UROWS
    units_per_tile = pl.cdiv(nunits, NS)
    c32 = c * HALF

    zero16 = jnp.zeros((16,), jnp.float32)

    def zb(i, _):
        zer_v[i, 0:16] = zero16
        zer_v[i, 16:32] = zero16
        return 0

    lax.fori_loop(0, UROWS, zb, 0)

    def zs(i, _):
        u = s + i * NS

        @pl.when(u < nunits)
        def _():
            pltpu.sync_copy(zer_v, aggr_sh.at[pl.ds(u * UROWS, UROWS)])

        return 0

    lax.fori_loop(0, units_per_tile, zs, 0)
    plsc.subcore_barrier()

    ebase = s * edges_per_tile
    emoff = (s // (NS // 2)) * (n_edges // 2)
    emcol = (s // (NS // 2)) * 64 + c32

    def block(b, _):
        base = ebase + b * EBLK
        pltpu.sync_copy(src_hbm.at[pl.ds(base, EBLK)], src_v)
        for j in range(NCHUNK):
            pltpu.sync_copy(dst_hbm.at[pl.ds(base + j * ECH, ECH)],
                            dst_v.at[j])

        noff = c * n_nodes

        def adj(j, _):
            sl = pl.ds(j * 16, 16)
            src_v[sl] = src_v[sl] + noff
            return 0

        lax.fori_loop(0, EBLK // 16, adj, 0)

        cps = [
            pltpu.async_copy(hm_hbm.at[src_v.at[pl.ds(j * ECH, ECH)]],
                             xj_v.at[pl.ds(j * ECH, ECH)], gsem)
            for j in range(NCHUNK)
        ]
        pltpu.sync_copy(
            em_hbm.at[pl.ds(base - emoff, EBLK), pl.ds(emcol, HALF)], em_v)
        for cp in cps:
            cp.wait()

        def rowfn(i, _):
            em_v[i, 0:16] = jnp.maximum(xj_v[i, 0:16] + em_v[i, 0:16], 0.0)
            em_v[i, 16:32] = jnp.maximum(xj_v[i, 16:32] + em_v[i, 16:32],
                                         0.0)
            return 0

        lax.fori_loop(0, EBLK, rowfn, 0)

        scs = [
            pltpu.async_copy(em_v.at[pl.ds(j * ECH, ECH)],
                             aggr_sh.at[dst_v.at[j]], ssem, add=True)
            for j in range(NCHUNK)
        ]
        for cp in scs:
            cp.wait()
        return 0

    lax.fori_loop(0, nblocks, block, 0)
    plsc.subcore_barrier()

    def co(i, _):
        u = s + i * NS

        @pl.when(u < nunits)
        def _():
            pltpu.sync_copy(aggr_sh.at[pl.ds(u * UROWS, UROWS)],
                            out_hbm.at[pl.ds(c * n_nodes + u * UROWS,
                                             UROWS)])

        return 0

    lax.fori_loop(0, units_per_tile, co, 0)


def _sc_edge_stage(hm_pad, em_pk, src, dst, n_nodes, n_edges):
    mesh = plsc.VectorSubcoreMesh(core_axis_name="c", subcore_axis_name="s",
                                  num_cores=NC, num_subcores=NS)
    body = functools.partial(_sc_edge_body, n_nodes, n_edges)
    return pl.kernel(
        body,
        out_type=jax.ShapeDtypeStruct((NC * n_nodes, HALF), jnp.float32),
        mesh=mesh,
        compiler_params=pltpu.CompilerParams(use_tc_tiling_on_sc=False),
        scratch_types=[
            pltpu.VMEM((EBLK,), jnp.int32),
            pltpu.VMEM((NCHUNK, ECH), jnp.int32),
            pltpu.VMEM((EBLK, HALF), jnp.float32),
            pltpu.VMEM((EBLK, HALF), jnp.float32),
            pltpu.VMEM((UROWS, HALF), jnp.float32),
            pltpu.VMEM_SHARED((n_nodes, HALF), jnp.float32),
            pltpu.SemaphoreType.DMA,
            pltpu.SemaphoreType.DMA,
        ],
    )(hm_pad, em_pk, src, dst)


def _head_body(nblocks, a_ref, b_ref, oh_ref, wu_ref, bu_ref, wo_ref,
               bo_ref, o_ref, sums_s, cnt_s):
    i = pl.program_id(0)
    a = a_ref[...]
    b = b_ref[...]
    blk = a.shape[0]
    u = jnp.dot(a, wu_ref[:HALF, :], preferred_element_type=jnp.float32)
    u = u + jnp.dot(b, wu_ref[HALF:, :], preferred_element_type=jnp.float32)
    u = jnp.maximum(u + bu_ref[...], 0.0)

    oh = oh_ref[...]
    ps = lax.dot_general(oh, u, (((0,), (0,)), ((), ())),
                         preferred_element_type=jnp.float32)
    pc = lax.dot_general(oh, jnp.ones((blk, 1), jnp.float32),
                         (((0,), (0,)), ((), ())),
                         preferred_element_type=jnp.float32)

    @pl.when(i == 0)
    def _():
        sums_s[...] = ps
        cnt_s[...] = pc

    @pl.when(i > 0)
    def _():
        sums_s[...] = sums_s[...] + ps
        cnt_s[...] = cnt_s[...] + pc

    @pl.when(i == nblocks - 1)
    def _():
        pooled = sums_s[...] / jnp.maximum(cnt_s[...], 1.0)
        o_ref[...] = jnp.dot(pooled, wo_ref[...],
                             preferred_element_type=jnp.float32) + bo_ref[...]


def _head(aggr_a, aggr_b, oh, wu, bu2d, wo, bo2d, blk):
    n = aggr_a.shape[0]
    nblocks = n // blk
    return pl.pallas_call(
        functools.partial(_head_body, nblocks),
        grid=(nblocks,),
        in_specs=[
            pl.BlockSpec((blk, HALF), lambda i: (i, 0)),
            pl.BlockSpec((blk, HALF), lambda i: (i, 0)),
            pl.BlockSpec((blk, G), lambda i: (i, 0)),
            pl.BlockSpec((2 * HALF, G), lambda i: (0, 0)),
            pl.BlockSpec((1, G), lambda i: (0, 0)),
            pl.BlockSpec((G, 1), lambda i: (0, 0)),
            pl.BlockSpec((1, 1), lambda i: (0, 0)),
        ],
        out_specs=pl.BlockSpec((G, 1), lambda i: (0, 0)),
        out_shape=jax.ShapeDtypeStruct((G, 1), jnp.float32),
        scratch_shapes=[
            pltpu.VMEM((G, G), jnp.float32),
            pltpu.VMEM((G, 1), jnp.float32),
        ],
    )(aggr_a, aggr_b, oh, wu, bu2d, wo, bo2d)


def kernel(x, edge_index, edge_attr, batch, Wn, bn, We, be, Wm, bm, Wu, bu,
           Wo, bo):
    n, _ = x.shape
    e = edge_attr.shape[0]
    h = Wn.shape[1]

    w1 = Wn @ Wm[:h]
    w2 = We @ Wm[h:]
    beff = (bn @ Wm[:h] + be @ Wm[h:] + bm).reshape(1, h)

    hm2 = _hm_stage(x, w1, beff, blk=10000).reshape(2 * n, HALF)
    em_pk = _em_stage(edge_attr, w2, blk=10000)

    aggr = _sc_edge_stage(hm2, em_pk, edge_index[0], edge_index[1],
                          n, e)

    aggr_a = aggr[:n]
    aggr_b = aggr[n:]
    oh = (batch[:, None] == jnp.arange(G, dtype=batch.dtype)[None, :]
          ).astype(jnp.float32)
    out = _head(aggr_a, aggr_b, oh, Wu, bu.reshape(1, h),
                Wo, bo.reshape(1, 1), blk=10000)
    return out.reshape(G)


# speedup vs baseline: 4.6127x; 1.5260x over previous
import functools

import jax
import jax.numpy as jnp
from jax import lax
from jax.experimental import pallas as pl
from jax.experimental.pallas import tpu as pltpu
from jax.experimental.pallas import tpu_sc as plsc

G = 64
HALF = 32
NS = 16
NC = 2
EBLK = 200
ECH = 40
NCHUNK = EBLK // ECH
UROWS = 80


def _hm_body(x_ref, w_ref, b_ref, o_ref):
    y = jnp.dot(x_ref[...], w_ref[...], preferred_element_type=jnp.float32)
    y = y + b_ref[...]
    o_ref[0] = y[:, :HALF]
    o_ref[1] = y[:, HALF:]


def _hm_stage(xmat, w, b, blk):
    n, d = xmat.shape
    h = w.shape[1]
    return pl.pallas_call(
        _hm_body,
        grid=(n // blk,),
        in_specs=[
            pl.BlockSpec((blk, d), lambda i: (i, 0)),
            pl.BlockSpec((d, h), lambda i: (0, 0)),
            pl.BlockSpec((1, h), lambda i: (0, 0)),
        ],
        out_specs=pl.BlockSpec((2, blk, HALF), lambda i: (0, i, 0)),
        out_shape=jax.ShapeDtypeStruct((2, n, HALF), jnp.float32),
    )(xmat, w, b)


def _em_body(a_ref, b_ref, w_ref, o_ref):
    ya = jnp.dot(a_ref[...], w_ref[...], preferred_element_type=jnp.float32)
    yb = jnp.dot(b_ref[...], w_ref[...], preferred_element_type=jnp.float32)
    o_ref[...] = jnp.concatenate([ya, yb], axis=1)


def _em_stage(ea, w, blk):
    e, d = ea.shape
    h = w.shape[1]
    off = (e // 2) // blk
    return pl.pallas_call(
        _em_body,
        grid=(off,),
        in_specs=[
            pl.BlockSpec((blk, d), lambda i: (i, 0)),
            pl.BlockSpec((blk, d), lambda i: (i + off, 0)),
            pl.BlockSpec((d, h), lambda i: (0, 0)),
        ],
        out_specs=pl.BlockSpec((blk, 128), lambda i: (i, 0)),
        out_shape=jax.ShapeDtypeStruct((e // 2, 128), jnp.float32),
    )(ea, ea, w)


def _sc_edge_body(n_nodes, n_edges, hm_hbm, em_hbm, srce_hbm, dst_hbm,
                  out_hbm, srca_v, srcb_v, dst4_v, xja_v, xjb_v, ema_v,
                  emb_v, zer_v, aggr_sh, si0, si1, sg0, sg1, ss0, ss1):
    c = lax.axis_index("c")
    s = lax.axis_index("s")
    edges_per_tile = n_edges // NS
    nblocks = edges_per_tile // EBLK
    nunits = n_nodes // UROWS
    units_per_tile = pl.cdiv(nunits, NS)
    c32 = c * HALF

    zero16 = jnp.zeros((16,), jnp.float32)

    def zb(i, _):
        zer_v[i, 0:16] = zero16
        zer_v[i, 16:32] = zero16
        return 0

    lax.fori_loop(0, UROWS, zb, 0)

    def zs(i, _):
        u = s + i * NS

        @pl.when(u < nunits)
        def _():
            pltpu.sync_copy(zer_v, aggr_sh.at[pl.ds(u * UROWS, UROWS)])

        return 0

    lax.fori_loop(0, units_per_tile, zs, 0)
    plsc.subcore_barrier()

    ebase = s * edges_per_tile
    emoff = (s // (NS // 2)) * (n_edges // 2)
    emcol = (s // (NS // 2)) * 64 + c32
    coff = c * n_edges

    srcb = [srca_v, srcb_v]
    xjb = [xja_v, xjb_v]
    emb = [ema_v, emb_v]
    sib = [si0, si1]
    sgb = [sg0, sg1]
    ssb = [ss0, ss1]

    def fire_idx(b, p, d):
        base = ebase + b * EBLK
        pltpu.async_copy(srce_hbm.at[pl.ds(coff + base, EBLK)],
                         srcb[p], sib[p])
        for j in range(NCHUNK):
            pltpu.async_copy(dst_hbm.at[pl.ds(base + j * ECH, ECH)],
                             dst4_v.at[d, j], sib[p])

    def wait_idx(b, p, d):
        base = ebase + b * EBLK
        pltpu.make_async_copy(srce_hbm.at[pl.ds(coff + base, EBLK)],
                              srcb[p], sib[p]).wait()
        for j in range(NCHUNK):
            pltpu.make_async_copy(dst_hbm.at[pl.ds(base + j * ECH, ECH)],
                                  dst4_v.at[d, j], sib[p]).wait()

    def fire_heavy(b, p):
        base = ebase + b * EBLK
        for j in range(NCHUNK):
            pltpu.async_copy(
                hm_hbm.at[srcb[p].at[pl.ds(j * ECH, ECH)]],
                xjb[p].at[pl.ds(j * ECH, ECH)], sgb[p])
        pltpu.async_copy(
            em_hbm.at[pl.ds(base - emoff, EBLK), pl.ds(emcol, HALF)],
            emb[p], sgb[p])

    def wait_heavy(b, p):
        base = ebase + b * EBLK
        for j in range(NCHUNK):
            pltpu.make_async_copy(
                hm_hbm.at[srcb[p].at[pl.ds(j * ECH, ECH)]],
                xjb[p].at[pl.ds(j * ECH, ECH)], sgb[p]).wait()
        pltpu.make_async_copy(
            em_hbm.at[pl.ds(base - emoff, EBLK), pl.ds(emcol, HALF)],
            emb[p], sgb[p]).wait()

    def compute(p):
        xj_v = xjb[p]
        em_v = emb[p]

        def rowfn(i, _):
            em_v[i, 0:16] = jnp.maximum(xj_v[i, 0:16] + em_v[i, 0:16], 0.0)
            em_v[i, 16:32] = jnp.maximum(xj_v[i, 16:32] + em_v[i, 16:32],
                                         0.0)
            return 0

        lax.fori_loop(0, EBLK, rowfn, 0)

    def fire_scatter(p, d):
        for j in range(NCHUNK):
            pltpu.async_copy(emb[p].at[pl.ds(j * ECH, ECH)],
                             aggr_sh.at[dst4_v.at[d, j]], ssb[p], add=True)

    def wait_scatter(p, d):
        for j in range(NCHUNK):
            pltpu.make_async_copy(emb[p].at[pl.ds(j * ECH, ECH)],
                                  aggr_sh.at[dst4_v.at[d, j]],
                                  ssb[p]).wait()

    fire_idx(0, 0, 0)
    fire_idx(1, 1, 1)
    wait_idx(0, 0, 0)
    fire_heavy(0, 0)

    wait_idx(1, 1, 1)
    fire_heavy(1, 1)
    wait_heavy(0, 0)
    fire_idx(2, 0, 2)
    compute(0)
    fire_scatter(0, 0)

    wait_scatter(0, 0)
    wait_idx(2, 0, 2)
    fire_heavy(2, 0)
    wait_heavy(1, 1)
    fire_idx(3, 1, 3)
    compute(1)
    fire_scatter(1, 1)

    def group(g, _):
        for q in range(4):
            b = 4 * g + 2 + q
            p = q & 1
            d = (2 + q) % 4
            wait_scatter(1 - p, (1 + q) % 4)

            @pl.when(b + 1 < nblocks)
            def _():
                wait_idx(b + 1, 1 - p, (3 + q) % 4)
                fire_heavy(b + 1, 1 - p)

            wait_heavy(b, p)

            @pl.when(b + 2 < nblocks)
            def _():
                fire_idx(b + 2, p, q)

            compute(p)
            fire_scatter(p, d)
        return 0

    lax.fori_loop(0, (nblocks - 2) // 4, group, 0)
    wait_scatter((nblocks - 1) & 1, (nblocks - 1) % 4)
    plsc.subcore_barrier()

    def co(i, _):
        u = s + i * NS

        @pl.when(u < nunits)
        def _():
            pltpu.sync_copy(aggr_sh.at[pl.ds(u * UROWS, UROWS)],
                            out_hbm.at[pl.ds(c * n_nodes + u * UROWS,
                                             UROWS)])

        return 0

    lax.fori_loop(0, units_per_tile, co, 0)


def _sc_edge_stage(hm_pad, em_pk, src, dst, n_nodes, n_edges):
    mesh = plsc.VectorSubcoreMesh(core_axis_name="c", subcore_axis_name="s",
                                  num_cores=NC, num_subcores=NS)
    body = functools.partial(_sc_edge_body, n_nodes, n_edges)
    return pl.kernel(
        body,
        out_type=jax.ShapeDtypeStruct((NC * n_nodes, HALF), jnp.float32),
        mesh=mesh,
        compiler_params=pltpu.CompilerParams(use_tc_tiling_on_sc=False),
        scratch_types=[
            pltpu.VMEM((EBLK,), jnp.int32),
            pltpu.VMEM((EBLK,), jnp.int32),
            pltpu.VMEM((4, NCHUNK, ECH), jnp.int32),
            pltpu.VMEM((EBLK, HALF), jnp.float32),
            pltpu.VMEM((EBLK, HALF), jnp.float32),
            pltpu.VMEM((EBLK, HALF), jnp.float32),
            pltpu.VMEM((EBLK, HALF), jnp.float32),
            pltpu.VMEM((UROWS, HALF), jnp.float32),
            pltpu.VMEM_SHARED((n_nodes, HALF), jnp.float32),
            pltpu.SemaphoreType.DMA,
            pltpu.SemaphoreType.DMA,
            pltpu.SemaphoreType.DMA,
            pltpu.SemaphoreType.DMA,
            pltpu.SemaphoreType.DMA,
            pltpu.SemaphoreType.DMA,
        ],
    )(hm_pad, em_pk, src, dst)


def _head_body(nblocks, a_ref, b_ref, oh_ref, wu_ref, bu_ref, wo_ref,
               bo_ref, o_ref, sums_s, cnt_s):
    i = pl.program_id(0)
    a = a_ref[...]
    b = b_ref[...]
    blk = a.shape[0]
    u = jnp.dot(a, wu_ref[:HALF, :], preferred_element_type=jnp.float32)
    u = u + jnp.dot(b, wu_ref[HALF:, :], preferred_element_type=jnp.float32)
    u = jnp.maximum(u + bu_ref[...], 0.0)

    oh = oh_ref[...]
    ps = lax.dot_general(oh, u, (((0,), (0,)), ((), ())),
                         preferred_element_type=jnp.float32)
    pc = lax.dot_general(oh, jnp.ones((blk, 1), jnp.float32),
                         (((0,), (0,)), ((), ())),
                         preferred_element_type=jnp.float32)

    @pl.when(i == 0)
    def _():
        sums_s[...] = ps
        cnt_s[...] = pc

    @pl.when(i > 0)
    def _():
        sums_s[...] = sums_s[...] + ps
        cnt_s[...] = cnt_s[...] + pc

    @pl.when(i == nblocks - 1)
    def _():
        pooled = sums_s[...] / jnp.maximum(cnt_s[...], 1.0)
        o_ref[...] = jnp.dot(pooled, wo_ref[...],
                             preferred_element_type=jnp.float32) + bo_ref[...]


def _head(aggr_a, aggr_b, oh, wu, bu2d, wo, bo2d, blk):
    n = aggr_a.shape[0]
    nblocks = n // blk
    return pl.pallas_call(
        functools.partial(_head_body, nblocks),
        grid=(nblocks,),
        in_specs=[
            pl.BlockSpec((blk, HALF), lambda i: (i, 0)),
            pl.BlockSpec((blk, HALF), lambda i: (i, 0)),
            pl.BlockSpec((blk, G), lambda i: (i, 0)),
            pl.BlockSpec((2 * HALF, G), lambda i: (0, 0)),
            pl.BlockSpec((1, G), lambda i: (0, 0)),
            pl.BlockSpec((G, 1), lambda i: (0, 0)),
            pl.BlockSpec((1, 1), lambda i: (0, 0)),
        ],
        out_specs=pl.BlockSpec((G, 1), lambda i: (0, 0)),
        out_shape=jax.ShapeDtypeStruct((G, 1), jnp.float32),
        scratch_shapes=[
            pltpu.VMEM((G, G), jnp.float32),
            pltpu.VMEM((G, 1), jnp.float32),
        ],
    )(aggr_a, aggr_b, oh, wu, bu2d, wo, bo2d)


def kernel(x, edge_index, edge_attr, batch, Wn, bn, We, be, Wm, bm, Wu, bu,
           Wo, bo):
    n, _ = x.shape
    e = edge_attr.shape[0]
    h = Wn.shape[1]

    w1 = Wn @ Wm[:h]
    w2 = We @ Wm[h:]
    beff = (bn @ Wm[:h] + be @ Wm[h:] + bm).reshape(1, h)

    hm2 = _hm_stage(x, w1, beff, blk=10000).reshape(2 * n, HALF)
    em_pk = _em_stage(edge_attr, w2, blk=10000)

    src = edge_index[0]
    src2 = jnp.concatenate([src, src + n])
    aggr = _sc_edge_stage(hm2, em_pk, src2, edge_index[1],
                          n, e)

    aggr_a = aggr[:n]
    aggr_b = aggr[n:]
    oh = (batch[:, None] == jnp.arange(G, dtype=batch.dtype)[None, :]
          ).astype(jnp.float32)
    out = _head(aggr_a, aggr_b, oh, Wu, bu.reshape(1, h),
                Wo, bo.reshape(1, 1), blk=10000)
    return out.reshape(G)


# speedup vs baseline: 6.6014x; 1.4311x over previous
import functools

import jax
import jax.numpy as jnp
from jax import lax
from jax.experimental import pallas as pl
from jax.experimental.pallas import tpu as pltpu
from jax.experimental.pallas import tpu_sc as plsc

G = 64
HALF = 32
NS = 16
NC = 2
EBLK = 200
ECH = 40
NCHUNK = EBLK // ECH
UROWS = 80


def _hm_body(x_ref, w_ref, b_ref, o_ref):
    y = jnp.dot(x_ref[...], w_ref[...], preferred_element_type=jnp.float32)
    y = y + b_ref[...]
    o_ref[0] = y[:, :HALF]
    o_ref[1] = y[:, HALF:]


def _hm_stage(xmat, w, b, blk):
    n, d = xmat.shape
    h = w.shape[1]
    return pl.pallas_call(
        _hm_body,
        grid=(n // blk,),
        in_specs=[
            pl.BlockSpec((blk, d), lambda i: (i, 0)),
            pl.BlockSpec((d, h), lambda i: (0, 0)),
            pl.BlockSpec((1, h), lambda i: (0, 0)),
        ],
        out_specs=pl.BlockSpec((2, blk, HALF), lambda i: (0, i, 0)),
        out_shape=jax.ShapeDtypeStruct((2, n, HALF), jnp.float32),
    )(xmat, w, b)


def _em_body(a_ref, b_ref, w_ref, o_ref):
    cdims = (((0,), (0,)), ((), ()))
    ya = lax.dot_general(a_ref[...], w_ref[...], cdims,
                         preferred_element_type=jnp.float32)
    yb = lax.dot_general(b_ref[...], w_ref[...], cdims,
                         preferred_element_type=jnp.float32)
    o_ref[...] = jnp.concatenate([ya, yb], axis=1)


def _em_stage(ea_t, w, blk):
    d, e = ea_t.shape
    h = w.shape[1]
    off = (e // 2) // blk
    return pl.pallas_call(
        _em_body,
        grid=(off,),
        in_specs=[
            pl.BlockSpec((d, blk), lambda i: (0, i)),
            pl.BlockSpec((d, blk), lambda i: (0, i + off)),
            pl.BlockSpec((d, h), lambda i: (0, 0)),
        ],
        out_specs=pl.BlockSpec((blk, 128), lambda i: (i, 0)),
        out_shape=jax.ShapeDtypeStruct((e // 2, 128), jnp.float32),
    )(ea_t, ea_t, w)


def _sc_edge_body(n_nodes, n_edges, hm_hbm, em_hbm, srce_hbm, dst_hbm,
                  out_hbm, srca_v, srcb_v, dst4_v, xja_v, xjb_v, ema_v,
                  emb_v, zer_v, aggr_sh, si0, si1, sg0, sg1, ss0, ss1):
    c = lax.axis_index("c")
    s = lax.axis_index("s")
    edges_per_tile = n_edges // NS
    nblocks = edges_per_tile // EBLK
    nunits = n_nodes // UROWS
    units_per_tile = pl.cdiv(nunits, NS)
    c32 = c * HALF

    zero16 = jnp.zeros((16,), jnp.float32)

    def zb(i, _):
        zer_v[i, 0:16] = zero16
        zer_v[i, 16:32] = zero16
        return 0

    lax.fori_loop(0, UROWS, zb, 0)

    def zs(i, _):
        u = s + i * NS

        @pl.when(u < nunits)
        def _():
            pltpu.sync_copy(zer_v, aggr_sh.at[pl.ds(u * UROWS, UROWS)])

        return 0

    lax.fori_loop(0, units_per_tile, zs, 0)
    plsc.subcore_barrier()

    ebase = s * edges_per_tile
    emoff = (s // (NS // 2)) * (n_edges // 2)
    emcol = (s // (NS // 2)) * 64 + c32
    coff = c * n_edges

    srcb = [srca_v, srcb_v]
    xjb = [xja_v, xjb_v]
    emb = [ema_v, emb_v]
    sib = [si0, si1]
    sgb = [sg0, sg1]
    ssb = [ss0, ss1]

    def fire_idx(b, p, d):
        base = ebase + b * EBLK
        pltpu.async_copy(srce_hbm.at[pl.ds(coff + base, EBLK)],
                         srcb[p], sib[p])
        for j in range(NCHUNK):
            pltpu.async_copy(dst_hbm.at[pl.ds(base + j * ECH, ECH)],
                             dst4_v.at[d, j], sib[p])

    def wait_idx(b, p, d):
        base = ebase + b * EBLK
        pltpu.make_async_copy(srce_hbm.at[pl.ds(coff + base, EBLK)],
                              srcb[p], sib[p]).wait()
        for j in range(NCHUNK):
            pltpu.make_async_copy(dst_hbm.at[pl.ds(base + j * ECH, ECH)],
                                  dst4_v.at[d, j], sib[p]).wait()

    def fire_heavy(b, p):
        base = ebase + b * EBLK
        for j in range(NCHUNK):
            pltpu.async_copy(
                hm_hbm.at[srcb[p].at[pl.ds(j * ECH, ECH)]],
                xjb[p].at[pl.ds(j * ECH, ECH)], sgb[p])
        pltpu.async_copy(
            em_hbm.at[pl.ds(base - emoff, EBLK), pl.ds(emcol, HALF)],
            emb[p], sgb[p])

    def wait_heavy(b, p):
        base = ebase + b * EBLK
        for j in range(NCHUNK):
            pltpu.make_async_copy(
                hm_hbm.at[srcb[p].at[pl.ds(j * ECH, ECH)]],
                xjb[p].at[pl.ds(j * ECH, ECH)], sgb[p]).wait()
        pltpu.make_async_copy(
            em_hbm.at[pl.ds(base - emoff, EBLK), pl.ds(emcol, HALF)],
            emb[p], sgb[p]).wait()

    def compute(p):
        xj_v = xjb[p]
        em_v = emb[p]

        @plsc.parallel_loop(0, EBLK, unroll=8)
        def rowfn(i):
            em_v[i, 0:16] = jnp.maximum(xj_v[i, 0:16] + em_v[i, 0:16], 0.0)
            em_v[i, 16:32] = jnp.maximum(xj_v[i, 16:32] + em_v[i, 16:32],
                                         0.0)

    def fire_scatter(p, d):
        for j in range(NCHUNK):
            pltpu.async_copy(emb[p].at[pl.ds(j * ECH, ECH)],
                             aggr_sh.at[dst4_v.at[d, j]], ssb[p], add=True)

    def wait_scatter(p, d):
        for j in range(NCHUNK):
            pltpu.make_async_copy(emb[p].at[pl.ds(j * ECH, ECH)],
                                  aggr_sh.at[dst4_v.at[d, j]],
                                  ssb[p]).wait()

    fire_idx(0, 0, 0)
    fire_idx(1, 1, 1)
    wait_idx(0, 0, 0)
    fire_heavy(0, 0)

    wait_idx(1, 1, 1)
    fire_heavy(1, 1)
    wait_heavy(0, 0)
    fire_idx(2, 0, 2)
    compute(0)
    fire_scatter(0, 0)

    wait_scatter(0, 0)
    wait_idx(2, 0, 2)
    fire_heavy(2, 0)
    wait_heavy(1, 1)
    fire_idx(3, 1, 3)
    compute(1)
    fire_scatter(1, 1)

    def group(g, _):
        for q in range(4):
            b = 4 * g + 2 + q
            p = q & 1
            d = (2 + q) % 4
            wait_scatter(1 - p, (1 + q) % 4)

            @pl.when(b + 1 < nblocks)
            def _():
                wait_idx(b + 1, 1 - p, (3 + q) % 4)
                fire_heavy(b + 1, 1 - p)

            wait_heavy(b, p)

            @pl.when(b + 2 < nblocks)
            def _():
                fire_idx(b + 2, p, q)

            compute(p)
            fire_scatter(p, d)
        return 0

    lax.fori_loop(0, (nblocks - 2) // 4, group, 0)
    wait_scatter((nblocks - 1) & 1, (nblocks - 1) % 4)
    plsc.subcore_barrier()

    def co(i, _):
        u = s + i * NS

        @pl.when(u < nunits)
        def _():
            pltpu.sync_copy(aggr_sh.at[pl.ds(u * UROWS, UROWS)],
                            out_hbm.at[pl.ds(c * n_nodes + u * UROWS,
                                             UROWS)])

        return 0

    lax.fori_loop(0, units_per_tile, co, 0)


def _sc_edge_stage(hm_pad, em_pk, src, dst, n_nodes, n_edges):
    mesh = plsc.VectorSubcoreMesh(core_axis_name="c", subcore_axis_name="s",
                                  num_cores=NC, num_subcores=NS)
    body = functools.partial(_sc_edge_body, n_nodes, n_edges)
    return pl.kernel(
        body,
        out_type=jax.ShapeDtypeStruct((NC * n_nodes, HALF), jnp.float32),
        mesh=mesh,
        compiler_params=pltpu.CompilerParams(use_tc_tiling_on_sc=False),
        scratch_types=[
            pltpu.VMEM((EBLK,), jnp.int32),
            pltpu.VMEM((EBLK,), jnp.int32),
            pltpu.VMEM((4, NCHUNK, ECH), jnp.int32),
            pltpu.VMEM((EBLK, HALF), jnp.float32),
            pltpu.VMEM((EBLK, HALF), jnp.float32),
            pltpu.VMEM((EBLK, HALF), jnp.float32),
            pltpu.VMEM((EBLK, HALF), jnp.float32),
            pltpu.VMEM((UROWS, HALF), jnp.float32),
            pltpu.VMEM_SHARED((n_nodes, HALF), jnp.float32),
            pltpu.SemaphoreType.DMA,
            pltpu.SemaphoreType.DMA,
            pltpu.SemaphoreType.DMA,
            pltpu.SemaphoreType.DMA,
            pltpu.SemaphoreType.DMA,
            pltpu.SemaphoreType.DMA,
        ],
    )(hm_pad, em_pk, src, dst)


def _head_body(nblocks, a_ref, b_ref, oh_ref, wu_ref, bu_ref, wo_ref,
               bo_ref, o_ref, sums_s, cnt_s):
    i = pl.program_id(0)
    a = a_ref[...]
    b = b_ref[...]
    blk = a.shape[0]
    u = jnp.dot(a, wu_ref[:HALF, :], preferred_element_type=jnp.float32)
    u = u + jnp.dot(b, wu_ref[HALF:, :], preferred_element_type=jnp.float32)
    u = jnp.maximum(u + bu_ref[...], 0.0)

    oh = oh_ref[...]
    ps = lax.dot_general(oh, u, (((0,), (0,)), ((), ())),
                         preferred_element_type=jnp.float32)
    pc = lax.dot_general(oh, jnp.ones((blk, 1), jnp.float32),
                         (((0,), (0,)), ((), ())),
                         preferred_element_type=jnp.float32)

    @pl.when(i == 0)
    def _():
        sums_s[...] = ps
        cnt_s[...] = pc

    @pl.when(i > 0)
    def _():
        sums_s[...] = sums_s[...] + ps
        cnt_s[...] = cnt_s[...] + pc

    @pl.when(i == nblocks - 1)
    def _():
        pooled = sums_s[...] / jnp.maximum(cnt_s[...], 1.0)
        o_ref[...] = jnp.dot(pooled, wo_ref[...],
                             preferred_element_type=jnp.float32) + bo_ref[...]


def _head(aggr_a, aggr_b, oh, wu, bu2d, wo, bo2d, blk):
    n = aggr_a.shape[0]
    nblocks = n // blk
    return pl.pallas_call(
        functools.partial(_head_body, nblocks),
        grid=(nblocks,),
        in_specs=[
            pl.BlockSpec((blk, HALF), lambda i: (i, 0)),
            pl.BlockSpec((blk, HALF), lambda i: (i, 0)),
            pl.BlockSpec((blk, G), lambda i: (i, 0)),
            pl.BlockSpec((2 * HALF, G), lambda i: (0, 0)),
            pl.BlockSpec((1, G), lambda i: (0, 0)),
            pl.BlockSpec((G, 1), lambda i: (0, 0)),
            pl.BlockSpec((1, 1), lambda i: (0, 0)),
        ],
        out_specs=pl.BlockSpec((G, 1), lambda i: (0, 0)),
        out_shape=jax.ShapeDtypeStruct((G, 1), jnp.float32),
        scratch_shapes=[
            pltpu.VMEM((G, G), jnp.float32),
            pltpu.VMEM((G, 1), jnp.float32),
        ],
    )(aggr_a, aggr_b, oh, wu, bu2d, wo, bo2d)


def kernel(x, edge_index, edge_attr, batch, Wn, bn, We, be, Wm, bm, Wu, bu,
           Wo, bo):
    n, _ = x.shape
    e = edge_attr.shape[0]
    h = Wn.shape[1]

    hi = lax.Precision.HIGHEST
    w1 = jnp.dot(Wn, Wm[:h], precision=hi)
    w2 = jnp.dot(We, Wm[h:], precision=hi)
    beff = (jnp.dot(bn, Wm[:h], precision=hi)
            + jnp.dot(be, Wm[h:], precision=hi) + bm).reshape(1, h)

    hm2 = _hm_stage(x, w1, beff, blk=10000).reshape(2 * n, HALF)
    em_pk = _em_stage(edge_attr.T, w2, blk=3200)

    src = edge_index[0]
    src2 = jnp.concatenate([src, src + n])
    aggr = _sc_edge_stage(hm2, em_pk, src2, edge_index[1],
                          n, e)

    aggr_a = aggr[:n]
    aggr_b = aggr[n:]
    oh = (batch[:, None] == jnp.arange(G, dtype=batch.dtype)[None, :]
          ).astype(jnp.float32)
    out = _head(aggr_a, aggr_b, oh, Wu, bu.reshape(1, h),
                Wo, bo.reshape(1, 1), blk=10000)
    return out.reshape(G)
